# baseline (speedup 1.0000x reference)
import functools

import jax
import jax.numpy as jnp
from jax import lax
from jax.experimental import pallas as pl
from jax.experimental.pallas import tpu as pltpu
from jax.experimental.pallas import tpu_sc as plsc

BATCH = 16384
EMBED = 16
GROUPS = 128 // EMBED


@functools.lru_cache(maxsize=None)
def _make_sc_gather():
    info = plsc.get_sparse_core_info()
    nc, ns = info.num_cores, info.num_subcores
    nw = nc * ns
    b_per_w = BATCH // nw
    mesh = plsc.VectorSubcoreMesh(core_axis_name="c", subcore_axis_name="s")

    @functools.partial(
        pl.kernel,
        mesh=mesh,
        out_type=jax.ShapeDtypeStruct((BATCH, 128), jnp.float32),
        scratch_types=[
            pltpu.VMEM((b_per_w,), jnp.int32),
            pltpu.VMEM((b_per_w, 128), jnp.float32),
            pltpu.SemaphoreType.DMA,
        ],
    )
    def gather_kernel(table_hbm, blk_hbm, out_hbm, blk_v, rows_v, sem):
        wid = lax.axis_index("s") * nc + lax.axis_index("c")
        base = wid * b_per_w
        pltpu.sync_copy(blk_hbm.at[pl.ds(base, b_per_w)], blk_v)
        pltpu.async_copy(table_hbm.at[blk_v], rows_v, sem).wait()
        pltpu.sync_copy(rows_v, out_hbm.at[pl.ds(base, b_per_w)])

    return gather_kernel


def _mlp_body(x_ref, off_ref, w1_ref, b1_ref, w2_ref, b2_ref, w3_ref, b3_ref, o_ref):
    off = off_ref[...]
    x = jnp.zeros((x_ref.shape[0], EMBED), jnp.float32)
    for s in range(GROUPS):
        x = x + jnp.where(off == s, x_ref[:, s * EMBED:(s + 1) * EMBED], 0.0)
    h = jnp.dot(x, w1_ref[...], preferred_element_type=jnp.float32)
    h = jnp.maximum(h + b1_ref[...], 0.0)
    h = jnp.dot(h, w2_ref[...], preferred_element_type=jnp.float32)
    h = jnp.maximum(h + b2_ref[...], 0.0)
    o = jnp.dot(h, w3_ref[...], preferred_element_type=jnp.float32)
    o_ref[...] = jax.nn.sigmoid(o + b3_ref[...])


def _tc_mlp(x, off, W1, b1, W2, b2, W3, b3):
    nb = 8
    blk = BATCH // nb
    return pl.pallas_call(
        _mlp_body,
        grid=(nb,),
        in_specs=[
            pl.BlockSpec((blk, 128), lambda i: (i, 0)),
            pl.BlockSpec((blk, 1), lambda i: (i, 0)),
            pl.BlockSpec((EMBED, 32), lambda i: (0, 0)),
            pl.BlockSpec((1, 32), lambda i: (0, 0)),
            pl.BlockSpec((32, 16), lambda i: (0, 0)),
            pl.BlockSpec((1, 16), lambda i: (0, 0)),
            pl.BlockSpec((16, 1), lambda i: (0, 0)),
            pl.BlockSpec((1, 1), lambda i: (0, 0)),
        ],
        out_specs=pl.BlockSpec((blk, 1), lambda i: (i, 0)),
        out_shape=jax.ShapeDtypeStruct((BATCH, 1), jnp.float32),
    )(x, off, W1, b1, W2, b2, W3, b3)


def kernel(inputs, table, W1, b1, W2, b2, W3, b3):
    idx = inputs.astype(jnp.int32)
    tab128 = table.reshape(table.shape[0] // GROUPS, 128)
    blk = idx // GROUPS
    off = (idx % GROUPS).reshape(BATCH, 1)
    rows = _make_sc_gather()(tab128, blk)
    return _tc_mlp(
        rows,
        off,
        W1,
        b1.reshape(1, 32),
        W2,
        b2.reshape(1, 16),
        W3,
        b3.reshape(1, 1),
    )


# speedup vs baseline: 1.5089x; 1.5089x over previous
import functools

import jax
import jax.numpy as jnp
from jax import lax
from jax.experimental import pallas as pl
from jax.experimental.pallas import tpu as pltpu
from jax.experimental.pallas import tpu_sc as plsc

BATCH = 16384
EMBED = 16
NUM_ROWS = 1000000
LANE_BLK = 8192
N_BLOCKS = -(-NUM_ROWS // LANE_BLK)
PACKED_ROWS = N_BLOCKS * (LANE_BLK // 8)


def _repack_body(x_ref, o_ref):
    x = x_ref[...]
    sub = LANE_BLK // 8
    parts = [jnp.transpose(x[:, s * sub:(s + 1) * sub]) for s in range(8)]
    o_ref[...] = jnp.concatenate(parts, axis=1)


def _tc_repack(tabT):
    return pl.pallas_call(
        _repack_body,
        grid=(N_BLOCKS,),
        in_specs=[pl.BlockSpec((EMBED, LANE_BLK), lambda i: (0, i))],
        out_specs=pl.BlockSpec((LANE_BLK // 8, 128), lambda i: (i, 0)),
        out_shape=jax.ShapeDtypeStruct((PACKED_ROWS, 128), jnp.float32),
    )(tabT)


@functools.lru_cache(maxsize=None)
def _make_sc_gather():
    info = plsc.get_sparse_core_info()
    nc, ns = info.num_cores, info.num_subcores
    nw = nc * ns
    b_per_w = BATCH // nw
    mesh = plsc.VectorSubcoreMesh(core_axis_name="c", subcore_axis_name="s")

    @functools.partial(
        pl.kernel,
        mesh=mesh,
        out_type=jax.ShapeDtypeStruct((BATCH, 128), jnp.float32),
        scratch_types=[
            pltpu.VMEM((b_per_w,), jnp.int32),
            pltpu.VMEM((b_per_w, 128), jnp.float32),
            pltpu.SemaphoreType.DMA,
        ],
    )
    def gather_kernel(packed_hbm, blk_hbm, out_hbm, blk_v, rows_v, sem):
        wid = lax.axis_index("s") * nc + lax.axis_index("c")
        base = wid * b_per_w
        pltpu.sync_copy(blk_hbm.at[pl.ds(base, b_per_w)], blk_v)
        pltpu.async_copy(packed_hbm.at[blk_v], rows_v, sem).wait()
        pltpu.sync_copy(rows_v, out_hbm.at[pl.ds(base, b_per_w)])

    return gather_kernel


def _mlp_body(x_ref, off_ref, w1_ref, b1_ref, w2_ref, b2_ref, w3_ref, b3_ref, o_ref):
    off = off_ref[...]
    x = jnp.zeros((x_ref.shape[0], EMBED), jnp.float32)
    for s in range(8):
        x = x + jnp.where(off == s, x_ref[:, s * EMBED:(s + 1) * EMBED], 0.0)
    h = jnp.dot(x, w1_ref[...], preferred_element_type=jnp.float32)
    h = jnp.maximum(h + b1_ref[...], 0.0)
    h = jnp.dot(h, w2_ref[...], preferred_element_type=jnp.float32)
    h = jnp.maximum(h + b2_ref[...], 0.0)
    o = jnp.dot(h, w3_ref[...], preferred_element_type=jnp.float32)
    o_ref[...] = jax.nn.sigmoid(o + b3_ref[...])


def _tc_mlp(x, off, W1, b1, W2, b2, W3, b3):
    nb = 8
    blk = BATCH // nb
    return pl.pallas_call(
        _mlp_body,
        grid=(nb,),
        in_specs=[
            pl.BlockSpec((blk, 128), lambda i: (i, 0)),
            pl.BlockSpec((blk, 1), lambda i: (i, 0)),
            pl.BlockSpec((EMBED, 32), lambda i: (0, 0)),
            pl.BlockSpec((1, 32), lambda i: (0, 0)),
            pl.BlockSpec((32, 16), lambda i: (0, 0)),
            pl.BlockSpec((1, 16), lambda i: (0, 0)),
            pl.BlockSpec((16, 1), lambda i: (0, 0)),
            pl.BlockSpec((1, 1), lambda i: (0, 0)),
        ],
        out_specs=pl.BlockSpec((blk, 1), lambda i: (i, 0)),
        out_shape=jax.ShapeDtypeStruct((BATCH, 1), jnp.float32),
    )(x, off, W1, b1, W2, b2, W3, b3)


def kernel(inputs, table, W1, b1, W2, b2, W3, b3):
    idx = inputs.astype(jnp.int32)
    packed = _tc_repack(table.T)
    sub = LANE_BLK // 8
    blk = (idx // LANE_BLK) * sub + idx % sub
    off = ((idx // sub) % 8).reshape(BATCH, 1)
    rows = _make_sc_gather()(packed, blk)
    return _tc_mlp(
        rows,
        off,
        W1,
        b1.reshape(1, 32),
        W2,
        b2.reshape(1, 16),
        W3,
        b3.reshape(1, 1),
    )
